# Initial kernel scaffold; baseline (speedup 1.0000x reference)
import jax
import jax.numpy as jnp
from jax.experimental import pallas as pl


def kernel(token_ids, table):
    raise NotImplementedError("write your pallas kernel here")


# speedup vs baseline: 2.8150x; 2.8150x over previous
import jax
import jax.numpy as jnp
from jax import lax
from jax.experimental import pallas as pl
from jax.experimental.pallas import tpu as pltpu
from jax.experimental.pallas import tpu_sc as plsc

D = 32
SEQ = 50
B = 16384
L = 16
NC, NS = 2, 16
NW = NC * NS
SENT_PER_W = B // NW
CHS = 64
NCHUNK = SENT_PER_W // CHS
TOK_PER_CHUNK = CHS * SEQ
GATHER = 128
NGATHER = TOK_PER_CHUNK // GATHER


def _body(ids_hbm, table_hbm, out_hbm, idx_v, rows_v, out_v, sem):
    wid = lax.axis_index("s") * NC + lax.axis_index("c")

    def chunk(c, carry):
        tok0 = wid * (SENT_PER_W * SEQ) + c * TOK_PER_CHUNK
        pltpu.sync_copy(ids_hbm.at[pl.ds(tok0, TOK_PER_CHUNK)], idx_v)
        descs = [
            pltpu.async_copy(
                table_hbm.at[idx_v.at[pl.ds(j * GATHER, GATHER)]],
                rows_v.at[pl.ds(j * GATHER, GATHER)],
                sem,
            )
            for j in range(NGATHER)
        ]
        for d_ in descs:
            d_.wait()

        def sent(s, carry2):
            base = s * SEQ
            acc0 = rows_v[base, pl.ds(0, L)]
            acc1 = rows_v[base, pl.ds(L, L)]
            for r in range(1, SEQ):
                acc0 = acc0 + rows_v[base + r, pl.ds(0, L)]
                acc1 = acc1 + rows_v[base + r, pl.ds(L, L)]
            out_v[s, pl.ds(0, L)] = acc0 * (1.0 / SEQ)
            out_v[s, pl.ds(L, L)] = acc1 * (1.0 / SEQ)
            return carry2

        lax.fori_loop(0, CHS, sent, 0)
        pltpu.sync_copy(out_v, out_hbm.at[pl.ds(wid * SENT_PER_W + c * CHS, CHS)])
        return carry

    lax.fori_loop(0, NCHUNK, chunk, 0)


def kernel(token_ids, table):
    ids = token_ids.astype(jnp.int32).reshape(B * SEQ)
    mesh = plsc.VectorSubcoreMesh(
        core_axis_name="c", subcore_axis_name="s", num_cores=NC, num_subcores=NS
    )
    f = pl.kernel(
        _body,
        out_type=jax.ShapeDtypeStruct((B, D), jnp.float32),
        mesh=mesh,
        scratch_types=[
            pltpu.VMEM((TOK_PER_CHUNK,), jnp.int32),
            pltpu.VMEM((TOK_PER_CHUNK, D), jnp.float32),
            pltpu.VMEM((CHS, D), jnp.float32),
            pltpu.SemaphoreType.DMA,
        ],
        compiler_params=pltpu.CompilerParams(use_tc_tiling_on_sc=False),
    )
    return f(ids, table)


# speedup vs baseline: 2.8203x; 1.0019x over previous
import jax
import jax.numpy as jnp
from jax import lax
from jax.experimental import pallas as pl
from jax.experimental.pallas import tpu as pltpu
from jax.experimental.pallas import tpu_sc as plsc

D = 32
SEQ = 50
B = 16384
L = 16
NC, NS = 2, 16
NW = NC * NS
SENT_PER_W = B // NW
CHS = 64
NCHUNK = SENT_PER_W // CHS
TOK_PER_CHUNK = CHS * SEQ
GATHER = 128
NGATHER = TOK_PER_CHUNK // GATHER


def _body(ids_hbm, table_hbm, out_hbm, idx_v, rows_v, out_v, sem):
    wid = lax.axis_index("s") * NC + lax.axis_index("c")

    def chunk(c, carry):
        tok0 = wid * (SENT_PER_W * SEQ) + c * TOK_PER_CHUNK
        pltpu.sync_copy(ids_hbm.at[pl.ds(tok0, TOK_PER_CHUNK)], idx_v)
        pltpu.async_copy(table_hbm.at[idx_v], rows_v, sem).wait()

        def sent(s, carry2):
            base = s * SEQ
            acc0 = rows_v[base, pl.ds(0, L)]
            acc1 = rows_v[base, pl.ds(L, L)]
            for r in range(1, SEQ):
                acc0 = acc0 + rows_v[base + r, pl.ds(0, L)]
                acc1 = acc1 + rows_v[base + r, pl.ds(L, L)]
            out_v[s, pl.ds(0, L)] = acc0 * (1.0 / SEQ)
            out_v[s, pl.ds(L, L)] = acc1 * (1.0 / SEQ)
            return carry2

        lax.fori_loop(0, CHS, sent, 0)
        pltpu.sync_copy(out_v, out_hbm.at[pl.ds(wid * SENT_PER_W + c * CHS, CHS)])
        return carry

    lax.fori_loop(0, NCHUNK, chunk, 0)


def kernel(token_ids, table):
    ids = token_ids.astype(jnp.int32).reshape(B * SEQ)
    mesh = plsc.VectorSubcoreMesh(
        core_axis_name="c", subcore_axis_name="s", num_cores=NC, num_subcores=NS
    )
    f = pl.kernel(
        _body,
        out_type=jax.ShapeDtypeStruct((B, D), jnp.float32),
        mesh=mesh,
        scratch_types=[
            pltpu.VMEM((TOK_PER_CHUNK,), jnp.int32),
            pltpu.VMEM((TOK_PER_CHUNK, D), jnp.float32),
            pltpu.VMEM((CHS, D), jnp.float32),
            pltpu.SemaphoreType.DMA,
        ],
        compiler_params=pltpu.CompilerParams(use_tc_tiling_on_sc=False),
    )
    return f(ids, table)
